# baseline (speedup 1.0000x reference)
import functools

import jax
import jax.numpy as jnp
from jax import lax
from jax.experimental import pallas as pl
from jax.experimental.pallas import tpu as pltpu
from jax.experimental.pallas import tpu_sc as plsc

BLOCK_R = 256
NC, NS, L = 2, 16, 16
NW = NC * NS
CHUNK = 128


def _sc_alpha_gather(t2, a_flat, R):
    b_per_w = R // NW
    CPR = b_per_w // CHUNK
    mesh = plsc.VectorSubcoreMesh(core_axis_name="c", subcore_axis_name="s")

    @functools.partial(
        pl.kernel,
        out_type=jax.ShapeDtypeStruct((R // CHUNK, CHUNK), jnp.float32),
        mesh=mesh,
        scratch_types=[
            pltpu.VMEM((CPR, CHUNK), jnp.int32),
            pltpu.VMEM((CPR, CHUNK), jnp.float32),
            pltpu.SemaphoreType.DMA,
        ],
    )
    def gather_kernel(t_hbm, a_hbm, at_hbm, t_v, at_v, sem):
        wid = lax.axis_index("s") * NC + lax.axis_index("c")
        row0 = wid * CPR
        pltpu.sync_copy(t_hbm.at[pl.ds(row0, CPR)], t_v)
        for j in range(CPR):
            pltpu.async_copy(a_hbm.at[t_v.at[j]], at_v.at[j], sem).wait()
        pltpu.sync_copy(at_v, at_hbm.at[pl.ds(row0, CPR)])

    return gather_kernel(t2, a_flat)


def kernel(inputs, targets, alpha):
    B, Q, N = inputs.shape
    R = B * Q
    x2 = inputs.reshape(R, N)
    t2 = targets.reshape(R // CHUNK, CHUNK)
    t3 = targets.reshape(R // BLOCK_R, 1, BLOCK_R)
    a_flat = alpha.reshape(N)

    at = _sc_alpha_gather(t2, a_flat, R)

    def w_body(x_ref, t_ref, w_ref):
        xb = x_ref[...]
        t = t_ref[0, 0, :]
        m = jnp.max(xb, axis=1, keepdims=True)
        s = jnp.sum(jnp.exp(xb - m), axis=1, keepdims=True)
        ids = jax.lax.broadcasted_iota(jnp.int32, xb.shape, 1)
        mask = ids == t[:, None]
        xt = jnp.sum(jnp.where(mask, xb, 0.0), axis=1, keepdims=True)
        logp = (xt - m) - jnp.log(s)
        p = jnp.exp(logp)
        q = 1.0 - p
        w_ref[...] = (q * q * logp).reshape(1, BLOCK_R // CHUNK, CHUNK)

    w = pl.pallas_call(
        w_body,
        grid=(R // BLOCK_R,),
        in_specs=[
            pl.BlockSpec((BLOCK_R, N), lambda i: (i, 0)),
            pl.BlockSpec((1, 1, BLOCK_R), lambda i: (i, 0, 0)),
        ],
        out_specs=pl.BlockSpec((1, BLOCK_R // CHUNK, CHUNK),
                               lambda i: (i, 0, 0)),
        out_shape=jax.ShapeDtypeStruct(
            (R // BLOCK_R, BLOCK_R // CHUNK, CHUNK), jnp.float32),
    )(x2, t3)
    w = w.reshape(R // CHUNK, CHUNK)

    def comb_body(at_ref, w_ref, o_ref):
        o_ref[0, 0] = jnp.sum(at_ref[...] * w_ref[...]) * (-1.0 / R)

    out = pl.pallas_call(
        comb_body,
        out_specs=pl.BlockSpec(memory_space=pltpu.SMEM),
        out_shape=jax.ShapeDtypeStruct((1, 1), jnp.float32),
    )(at, w)
    return out[0, 0]
